# Initial kernel scaffold; baseline (speedup 1.0000x reference)
import jax
import jax.numpy as jnp
from jax.experimental import pallas as pl


def kernel(loc_data, conf_data, prior_box, targets):
    raise NotImplementedError("write your pallas kernel here")


# speedup vs baseline: 39.1787x; 39.1787x over previous
import jax
import jax.numpy as jnp
from jax.experimental import pallas as pl
from jax.experimental.pallas import tpu as pltpu

_NCLS = 21
_NOBJ = 8
_THRESH = 0.5
_NEGPOS = 3
_V0, _V1 = 0.1, 0.2


def _mbl_kernel(tgt_ref, pri_ref, loc_ref, conf_ref, ll_ref, lc_ref, nn_ref):
    pi = pl.program_id(0)
    nsteps = pl.num_programs(0)
    Bc = tgt_ref.shape[0]
    Pn = pri_ref.shape[1]
    f32 = jnp.float32
    i32 = jnp.int32

    @pl.when(pi == 0)
    def _init():
        ll_ref[0, 0] = 0.0
        lc_ref[0, 0] = 0.0
        nn_ref[0, 0] = 0.0

    px = pri_ref[0:1, :]
    py = pri_ref[1:2, :]
    pw = pri_ref[2:3, :]
    ph = pri_ref[3:4, :]
    PX1 = px - pw * 0.5
    PY1 = py - ph * 0.5
    PX2 = px + pw * 0.5
    PY2 = py + ph * 0.5
    area_p = (PX2 - PX1) * (PY2 - PY1)

    t = tgt_ref[...]

    lane = jax.lax.broadcasted_iota(i32, (1, Pn), 1)

    tx1s, ty1s, tx2s, ty2s, tls = [], [], [], [], []
    bp_idx = []
    bto = None
    bti = None
    for j in range(_NOBJ):
        tx1 = t[:, 5 * j + 0:5 * j + 1]
        ty1 = t[:, 5 * j + 1:5 * j + 2]
        tx2 = t[:, 5 * j + 2:5 * j + 3]
        ty2 = t[:, 5 * j + 3:5 * j + 4]
        tl = t[:, 5 * j + 4:5 * j + 5]
        tx1s.append(tx1); ty1s.append(ty1); tx2s.append(tx2); ty2s.append(ty2)
        tls.append(tl)
        iw = jnp.maximum(jnp.minimum(tx2, PX2) - jnp.maximum(tx1, PX1), 0.0)
        ih = jnp.maximum(jnp.minimum(ty2, PY2) - jnp.maximum(ty1, PY1), 0.0)
        inter = iw * ih
        area_t = (tx2 - tx1) * (ty2 - ty1)
        iou = inter / (area_t + area_p - inter)
        m = jnp.max(iou, axis=1, keepdims=True)
        idx = jnp.min(jnp.where(iou == m, lane, Pn), axis=1, keepdims=True)
        bp_idx.append(idx)
        if j == 0:
            bto = iou
            bti = jnp.zeros((Bc, Pn), i32)
        else:
            upd = iou > bto
            bti = jnp.where(upd, j, bti)
            bto = jnp.maximum(bto, iou)

    for j in range(_NOBJ):
        mask = lane == bp_idx[j]
        bto = jnp.where(mask, 2.0, bto)
        bti = jnp.where(mask, j, bti)

    lbl = jnp.zeros((Bc, Pn), f32)
    mx1 = jnp.zeros((Bc, Pn), f32)
    my1 = jnp.zeros((Bc, Pn), f32)
    mx2 = jnp.zeros((Bc, Pn), f32)
    my2 = jnp.zeros((Bc, Pn), f32)
    for j in range(_NOBJ):
        sel = bti == j
        lbl = jnp.where(sel, tls[j], lbl)
        mx1 = jnp.where(sel, tx1s[j], mx1)
        my1 = jnp.where(sel, ty1s[j], my1)
        mx2 = jnp.where(sel, tx2s[j], mx2)
        my2 = jnp.where(sel, ty2s[j], my2)
    conf_t = jnp.where(bto < _THRESH, 0, lbl.astype(i32) + 1)
    pos = conf_t > 0
    posf = pos.astype(f32)

    g0 = ((mx1 + mx2) * 0.5 - px) / (_V0 * pw)
    g1 = ((my1 + my2) * 0.5 - py) / (_V0 * ph)
    g2 = jnp.log((mx2 - mx1) / pw) / _V1
    g3 = jnp.log((my2 - my1) / ph) / _V1
    ll_acc = 0.0
    for c, g in enumerate((g0, g1, g2, g3)):
        d = loc_ref[:, c, :] - g
        ad = jnp.abs(d)
        sl1 = jnp.where(ad < 1.0, 0.5 * d * d, ad - 0.5)
        ll_acc = ll_acc + jnp.sum(sl1 * posf)

    m = conf_ref[:, 0, :]
    for c in range(1, _NCLS):
        m = jnp.maximum(m, conf_ref[:, c, :])
    s = jnp.zeros((Bc, Pn), f32)
    cg = jnp.zeros((Bc, Pn), f32)
    for c in range(_NCLS):
        cc = conf_ref[:, c, :]
        s = s + jnp.exp(cc - m)
        cg = jnp.where(conf_t == c, cc, cg)
    ce = jnp.log(s) + m - cg

    np_row = jnp.sum(pos.astype(i32), axis=1, keepdims=True)
    k = jnp.minimum(_NEGPOS * np_row, Pn - 1)

    v = jnp.where(pos, 0.0, ce)
    vb = jax.lax.bitcast_convert_type(v, i32)
    cand = jnp.zeros((Bc, 1), i32)
    for bit in range(30, -1, -1):
        test = cand | (1 << bit)
        cnt = jnp.sum((vb >= test).astype(i32), axis=1, keepdims=True)
        cand = jnp.where(cnt >= k, test, cand)
    thr = jax.lax.bitcast_convert_type(cand, f32)
    gt = v > thr
    cnt_gt = jnp.sum(gt.astype(i32), axis=1, keepdims=True)
    sum_gt = jnp.sum(jnp.where(gt, v, 0.0), axis=1, keepdims=True)
    top = sum_gt + (k - cnt_gt).astype(f32) * thr

    lc_acc = jnp.sum(ce * posf) + jnp.sum(top)
    n_acc = jnp.sum(np_row).astype(f32)

    ll_ref[0, 0] += ll_acc
    lc_ref[0, 0] += lc_acc
    nn_ref[0, 0] += n_acc

    @pl.when(pi == nsteps - 1)
    def _fin():
        n = jnp.maximum(nn_ref[0, 0], 1.0)
        ll_ref[0, 0] = ll_ref[0, 0] / n
        lc_ref[0, 0] = lc_ref[0, 0] / n


def kernel(loc_data, conf_data, prior_box, targets):
    B, Pn, _ = loc_data.shape
    locT = jnp.transpose(loc_data, (0, 2, 1))
    confT = jnp.transpose(conf_data, (0, 2, 1))
    priT = jnp.transpose(prior_box, (1, 0))
    tgt = targets.reshape(B, -1)
    Bc = 8
    grid = B // Bc
    ll, lc, _nn = pl.pallas_call(
        _mbl_kernel,
        grid=(grid,),
        in_specs=[
            pl.BlockSpec((Bc, tgt.shape[1]), lambda i: (i, 0)),
            pl.BlockSpec((4, Pn), lambda i: (0, 0)),
            pl.BlockSpec((Bc, 4, Pn), lambda i: (i, 0, 0)),
            pl.BlockSpec((Bc, _NCLS, Pn), lambda i: (i, 0, 0)),
        ],
        out_specs=[
            pl.BlockSpec(memory_space=pltpu.SMEM),
            pl.BlockSpec(memory_space=pltpu.SMEM),
            pl.BlockSpec(memory_space=pltpu.SMEM),
        ],
        out_shape=[
            jax.ShapeDtypeStruct((1, 1), jnp.float32),
            jax.ShapeDtypeStruct((1, 1), jnp.float32),
            jax.ShapeDtypeStruct((1, 1), jnp.float32),
        ],
    )(tgt, priT, locT, confT)
    return ll[0, 0], lc[0, 0]
